# baseline (speedup 1.0000x reference)
import functools

import jax
import jax.numpy as jnp
import numpy as np
from jax import lax
from jax.experimental import pallas as pl
from jax.experimental.pallas import tpu as pltpu
from jax.experimental.pallas import tpu_sc as plsc

N_NODES = 25000
N_EDGES = 400000
IN_DIM = 128
HID = 64
HEADS = 4
DH = 16
NGRAPH = 16

SC_CORES = 2
SC_SUBCORES = 16
NW = SC_CORES * SC_SUBCORES

EDGE_PAD = 409600
PER_W = EDGE_PAD // NW
CHUNK = 128
NCHUNK = PER_W // CHUNK
SEG_PAD = 25088
ROWS_PER_TILE = SEG_PAD // SC_SUBCORES
TW = 128
SEG_HALF = SEG_PAD // 2
HALF_PER_TILE = SEG_HALF // SC_SUBCORES
ACC_ROWS = 12672
ZROWS_PER_TILE = ACC_ROWS // SC_SUBCORES
SCHUNK = 80
NCH_SC = EDGE_PAD // (SC_SUBCORES * SCHUNK)


def _tc_linear(x, W, b, act):
    M, K = x.shape
    N = W.shape[1]
    BM = 1000

    def body(x_ref, w_ref, b_ref, o_ref):
        acc = jnp.dot(x_ref[...], w_ref[...], preferred_element_type=jnp.float32)
        acc = acc + b_ref[...]
        if act == "relu":
            acc = jnp.maximum(acc, 0.0)
        o_ref[...] = acc

    return pl.pallas_call(
        body,
        grid=(M // BM,),
        in_specs=[
            pl.BlockSpec((BM, K), lambda i: (i, 0)),
            pl.BlockSpec((K, N), lambda i: (0, 0)),
            pl.BlockSpec((1, N), lambda i: (0, 0)),
        ],
        out_specs=pl.BlockSpec((BM, N), lambda i: (i, 0)),
        out_shape=jax.ShapeDtypeStruct((M, N), jnp.float32),
    )(x, W, b.reshape(1, -1))


def _tc_qkv(x, Wq, bq, Wk, bk, Wv, bv):
    M, K = x.shape
    BM = 1000

    def body(x_ref, wq_ref, bq_ref, wk_ref, bk_ref, wv_ref, bv_ref,
             qt_ref, kvt_ref):
        xb = x_ref[...]
        q = jnp.dot(xb, wq_ref[...], preferred_element_type=jnp.float32) + bq_ref[...]
        k = jnp.dot(xb, wk_ref[...], preferred_element_type=jnp.float32) + bk_ref[...]
        v = jnp.dot(xb, wv_ref[...], preferred_element_type=jnp.float32) + bv_ref[...]
        qt_ref[...] = jnp.concatenate([q, jnp.zeros((BM, HID), jnp.float32)], axis=1)
        kvt_ref[...] = jnp.concatenate([k, v], axis=1)

    wspec = pl.BlockSpec((K, HID), lambda i: (0, 0))
    bspec = pl.BlockSpec((1, HID), lambda i: (0, 0))
    ospec = pl.BlockSpec((BM, TW), lambda i: (i, 0))
    return pl.pallas_call(
        body,
        grid=(M // BM,),
        in_specs=[pl.BlockSpec((BM, K), lambda i: (i, 0)),
                  wspec, bspec, wspec, bspec, wspec, bspec],
        out_specs=[ospec, ospec],
        out_shape=[jax.ShapeDtypeStruct((M, TW), jnp.float32)] * 2,
    )(x, Wq, bq.reshape(1, -1), Wk, bk.reshape(1, -1), Wv, bv.reshape(1, -1))


def _tc_logits(qg, kvg, head_sum_pad, p_scaled):
    BE = 2048
    grid = EDGE_PAD // BE

    def body(q_ref, m_ref, s_ref, p_ref, lg_ref, gm_ref):
        lg = jnp.dot(q_ref[...] * m_ref[...], s_ref[...],
                     preferred_element_type=jnp.float32) * p_ref[...]
        lg_ref[...] = lg
        bm = jnp.max(lg, axis=0, keepdims=True)

        @pl.when(pl.program_id(0) == 0)
        def _():
            gm_ref[...] = jnp.full((1, HEADS), -jnp.inf, jnp.float32)

        gm_ref[...] = jnp.maximum(gm_ref[...], bm)

    return pl.pallas_call(
        body,
        grid=(grid,),
        in_specs=[
            pl.BlockSpec((BE, TW), lambda i: (i, 0)),
            pl.BlockSpec((BE, TW), lambda i: (i, 0)),
            pl.BlockSpec((TW, HEADS), lambda i: (0, 0)),
            pl.BlockSpec((1, HEADS), lambda i: (0, 0)),
        ],
        out_specs=[pl.BlockSpec((BE, HEADS), lambda i: (i, 0)),
                   pl.BlockSpec((1, HEADS), lambda i: (0, 0))],
        out_shape=[jax.ShapeDtypeStruct((EDGE_PAD, HEADS), jnp.float32),
                   jax.ShapeDtypeStruct((1, HEADS), jnp.float32)],
    )(qg, kvg, head_sum_pad, p_scaled)


def _tc_msg(logits, gmax, mout, head_rep):
    BE = 2048
    grid = EDGE_PAD // BE

    def body(lg_ref, gm_ref, m_ref, r_ref, o_ref):
        ex = jnp.exp(lg_ref[...] - gm_ref[...])
        rows = pl.program_id(0) * BE + lax.broadcasted_iota(jnp.int32, (BE, HEADS), 0)
        ex = jnp.where(rows < N_EDGES, ex, 0.0)
        exb = jnp.dot(ex, r_ref[...], preferred_element_type=jnp.float32)
        msg = m_ref[...][:, HID:] * exb
        o_ref[...] = jnp.concatenate(
            [msg, ex, jnp.zeros((BE, TW - HID - HEADS), jnp.float32)], axis=1)

    return pl.pallas_call(
        body,
        grid=(grid,),
        in_specs=[
            pl.BlockSpec((BE, HEADS), lambda i: (i, 0)),
            pl.BlockSpec((1, HEADS), lambda i: (0, 0)),
            pl.BlockSpec((BE, TW), lambda i: (i, 0)),
            pl.BlockSpec((HEADS, HID), lambda i: (0, 0)),
        ],
        out_specs=pl.BlockSpec((BE, TW), lambda i: (i, 0)),
        out_shape=jax.ShapeDtypeStruct((EDGE_PAD, TW), jnp.float32),
    )(logits, gmax, mout, head_rep)


def _tc_combine(parts, x_prev_pad, Wa, ba, skip, sel_msg, sel_den):
    BM = SEG_HALF // 16

    def body(p_ref, x_ref, wa_ref, ba_ref, sk_ref, pm_ref, pd_ref, o_ref):
        s = p_ref[0]
        num = jnp.dot(s, pm_ref[...], preferred_element_type=jnp.float32)
        den = jnp.dot(s, pd_ref[...], preferred_element_type=jnp.float32)
        attn = num / (den + 1e-16)
        g = jax.nn.gelu(attn)
        o = jnp.dot(g, wa_ref[...], preferred_element_type=jnp.float32) + ba_ref[...]
        a = jax.nn.sigmoid(sk_ref[0, 0])
        o_ref[...] = a * o + (1.0 - a) * x_ref[...]

    return pl.pallas_call(
        body,
        grid=(SC_CORES, 16),
        in_specs=[
            pl.BlockSpec((1, BM, TW), lambda c, i: (c, i, 0)),
            pl.BlockSpec((BM, HID), lambda c, i: (c * 16 + i, 0)),
            pl.BlockSpec((HID, HID), lambda c, i: (0, 0)),
            pl.BlockSpec((1, HID), lambda c, i: (0, 0)),
            pl.BlockSpec((1, 1), lambda c, i: (0, 0)),
            pl.BlockSpec((TW, HID), lambda c, i: (0, 0)),
            pl.BlockSpec((TW, HID), lambda c, i: (0, 0)),
        ],
        out_specs=pl.BlockSpec((BM, HID), lambda c, i: (c * 16 + i, 0)),
        out_shape=jax.ShapeDtypeStruct((SEG_PAD, HID), jnp.float32),
    )(parts, x_prev_pad, Wa, ba.reshape(1, -1), skip.reshape(1, 1),
      sel_msg, sel_den)


def _tc_pool(xc, xv, bc, bv, Wl, bl):
    BM = 1000
    grid = N_NODES // BM

    def body(xc_ref, xv_ref, bc_ref, bv_ref, wl_ref, bl_ref, o_ref, accc, accv):
        pid = pl.program_id(0)

        @pl.when(pid == 0)
        def _():
            accc[...] = jnp.full((NGRAPH, HID), -jnp.inf, jnp.float32)
            accv[...] = jnp.full((NGRAPH, HID), -jnp.inf, jnp.float32)

        xcb = xc_ref[...]
        xvb = xv_ref[...]
        bcb = bc_ref[...]
        bvb = bv_ref[...]
        for g in range(NGRAPH):
            mc = jnp.where(bcb == g, xcb, -jnp.inf)
            mv = jnp.where(bvb == g, xvb, -jnp.inf)
            accc[g:g + 1, :] = jnp.maximum(accc[g:g + 1, :],
                                           jnp.max(mc, axis=0, keepdims=True))
            accv[g:g + 1, :] = jnp.maximum(accv[g:g + 1, :],
                                           jnp.max(mv, axis=0, keepdims=True))

        @pl.when(pid == grid - 1)
        def _():
            cp = accc[...]
            vp = accv[...]
            cp = jnp.where(jnp.isfinite(cp), cp, 0.0)
            vp = jnp.where(jnp.isfinite(vp), vp, 0.0)
            feat = jnp.concatenate([vp, cp], axis=1)
            lg = jnp.dot(feat, wl_ref[...], preferred_element_type=jnp.float32) + bl_ref[...]
            m = jnp.max(lg, axis=1, keepdims=True)
            e = jnp.exp(lg - m)
            p = e / jnp.sum(e, axis=1, keepdims=True)
            o_ref[...] = jnp.concatenate(
                [p, jnp.zeros((NGRAPH, 128 - 2), jnp.float32)], axis=1)

    return pl.pallas_call(
        body,
        grid=(grid,),
        in_specs=[
            pl.BlockSpec((BM, HID), lambda i: (i, 0)),
            pl.BlockSpec((BM, HID), lambda i: (i, 0)),
            pl.BlockSpec((BM, 1), lambda i: (i, 0)),
            pl.BlockSpec((BM, 1), lambda i: (i, 0)),
            pl.BlockSpec((2 * HID, 2), lambda i: (0, 0)),
            pl.BlockSpec((1, 2), lambda i: (0, 0)),
        ],
        out_specs=pl.BlockSpec((NGRAPH, 128), lambda i: (0, 0)),
        out_shape=jax.ShapeDtypeStruct((NGRAPH, 128), jnp.float32),
        scratch_shapes=[pltpu.VMEM((NGRAPH, HID), jnp.float32),
                        pltpu.VMEM((NGRAPH, HID), jnp.float32)],
    )(xc, xv, bc, bv, Wl, bl.reshape(1, -1))


_sc_cache = {}


def _sc_mesh():
    return plsc.VectorSubcoreMesh(core_axis_name="c", subcore_axis_name="s",
                                  num_cores=SC_CORES,
                                  num_subcores=SC_SUBCORES)


def _sc_gather3(qt, kvt, di, si):
    if "gather3" not in _sc_cache:
        @functools.partial(
            pl.kernel,
            out_type=[jax.ShapeDtypeStruct((EDGE_PAD, TW), jnp.float32)] * 2,
            mesh=_sc_mesh(),
            scratch_types=[
                pltpu.VMEM((PER_W,), jnp.int32),
                pltpu.VMEM((PER_W,), jnp.int32),
                [pltpu.VMEM((CHUNK, TW), jnp.float32) for _ in range(2)],
                [pltpu.VMEM((CHUNK, TW), jnp.float32) for _ in range(2)],
                [pltpu.SemaphoreType.DMA for _ in range(2)],
                [pltpu.SemaphoreType.DMA for _ in range(2)],
            ],
        )
        def body(qt_h, kvt_h, di_h, si_h, qg_h, kvg_h, di_all, si_all, qr, kr,
                 gsem, osem):
            wid = lax.axis_index("s") * SC_CORES + lax.axis_index("c")
            base0 = wid * PER_W

            pltpu.sync_copy(di_h.at[pl.ds(base0, PER_W)], di_all)
            pltpu.sync_copy(si_h.at[pl.ds(base0, PER_W)], si_all)

            def issue(cur, b):
                off = cur * CHUNK
                pltpu.async_copy(
                    qt_h.at[di_all.at[pl.ds(off, CHUNK)]], qr[b], gsem[b])
                pltpu.async_copy(
                    kvt_h.at[si_all.at[pl.ds(off, CHUNK)]], kr[b], gsem[b])

            def process(cur, b):
                off = cur * CHUNK
                pltpu.make_async_copy(
                    qt_h.at[di_all.at[pl.ds(off, CHUNK)]], qr[b],
                    gsem[b]).wait()
                pltpu.make_async_copy(
                    kvt_h.at[si_all.at[pl.ds(off, CHUNK)]], kr[b],
                    gsem[b]).wait()
                base = base0 + cur * CHUNK
                wq = pltpu.async_copy(qr[b], qg_h.at[pl.ds(base, CHUNK)],
                                      osem[b])
                wk = pltpu.async_copy(kr[b], kvg_h.at[pl.ds(base, CHUNK)],
                                      osem[b])
                return wq, wk

            issue(0, 0)
            issue(1, 1)

            def pair(i, carry):
                g = i * 2
                for b in range(2):
                    cur = g + b
                    wq, wk = process(cur, b)
                    wq.wait()
                    wk.wait()
                    issue(cur + 2, b)
                return carry

            lax.fori_loop(0, (NCHUNK - 2) // 2, pair, 0)
            for b in range(2):
                wq, wk = process(NCHUNK - 2 + b, b)
                wq.wait()
                wk.wait()

        _sc_cache["gather3"] = body
    return _sc_cache["gather3"](qt, kvt, di, si)


def _sc_scatter(msg_ext, di, zeros_acc):
    if "scatter" not in _sc_cache:
        @functools.partial(
            pl.kernel,
            out_type=jax.ShapeDtypeStruct((SC_CORES * SEG_HALF, TW), jnp.float32),
            mesh=_sc_mesh(),
            scratch_types=[
                [pltpu.VMEM((SCHUNK,), jnp.int32) for _ in range(2)],
                [pltpu.VMEM((SCHUNK,), jnp.int32) for _ in range(2)],
                [pltpu.VMEM((SCHUNK, TW), jnp.float32) for _ in range(2)],
                pltpu.VMEM_SHARED((ACC_ROWS, TW), jnp.float32),
                [pltpu.SemaphoreType.DMA for _ in range(2)],
                [pltpu.SemaphoreType.DMA for _ in range(2)],
            ],
        )
        def body(msg_h, di_h, zeros_h, out_h, di_v, dloc_v, msg_v, acc_sh,
                 rsem, ssem):
            cid = lax.axis_index("c")
            sid = lax.axis_index("s")
            zrow0 = sid * ZROWS_PER_TILE
            lo = cid * SEG_HALF
            base0 = sid * NCH_SC * SCHUNK

            pltpu.sync_copy(zeros_h.at[pl.ds(zrow0, ZROWS_PER_TILE)],
                            acc_sh.at[pl.ds(zrow0, ZROWS_PER_TILE)])
            plsc.subcore_barrier()

            def issue(cur, b):
                base = base0 + cur * SCHUNK
                pltpu.async_copy(di_h.at[pl.ds(base, SCHUNK)], di_v[b], rsem[b])
                pltpu.async_copy(msg_h.at[pl.ds(base, SCHUNK)], msg_v[b], rsem[b])

            def process(cur, b):
                base = base0 + cur * SCHUNK
                pltpu.make_async_copy(di_h.at[pl.ds(base, SCHUNK)], di_v[b],
                                      rsem[b]).wait()
                pltpu.make_async_copy(msg_h.at[pl.ds(base, SCHUNK)], msg_v[b],
                                      rsem[b]).wait()
                for j in range(SCHUNK // 16):
                    sl = pl.ds(j * 16, 16)
                    d = di_v[b][sl] - lo
                    ok = (d >= 0) & (d < SEG_HALF)
                    dloc_v[b][sl] = jnp.where(ok, d, SEG_HALF)
                return pltpu.async_copy(msg_v[b], acc_sh.at[dloc_v[b]],
                                        ssem[b], add=True)

            issue(0, 0)
            issue(1, 1)

            def pair(i, carry):
                g = i * 2
                for b in range(2):
                    cur = g + b
                    w = process(cur, b)
                    w.wait()
                    issue(cur + 2, b)
                return carry

            lax.fori_loop(0, (NCH_SC - 2) // 2, pair, 0)
            process(NCH_SC - 2, 0).wait()
            process(NCH_SC - 1, 1).wait()
            plsc.subcore_barrier()

            row0 = sid * HALF_PER_TILE
            pltpu.sync_copy(acc_sh.at[pl.ds(row0, HALF_PER_TILE)],
                            out_h.at[pl.ds(cid * SEG_HALF + row0, HALF_PER_TILE)])

        _sc_cache["scatter"] = body
    return _sc_cache["scatter"](msg_ext, di, zeros_acc).reshape(
        SC_CORES, SEG_HALF, TW)


def _blockdiag(rel):
    return jax.scipy.linalg.block_diag(*[rel[h] for h in range(HEADS)])


def _pad_idx(idx):
    return jnp.concatenate([idx, jnp.zeros((EDGE_PAD - N_EDGES,), jnp.int32)])


def kernel(x_constraint, x_variable, edge_index_vc, edge_index_cv,
           batch_constraint, batch_variable, params):
    f32 = jnp.float32

    eye_h = np.zeros((HID, HEADS), np.float32)
    for h in range(HEADS):
        eye_h[h * DH:(h + 1) * DH, h] = 1.0
    head_sum_pad = jnp.asarray(
        np.concatenate([eye_h, np.zeros((TW - HID, HEADS), np.float32)]))
    head_rep = jnp.asarray(eye_h.T)
    selm = np.zeros((TW, HID), np.float32)
    selm[:HID, :HID] = np.eye(HID, dtype=np.float32)
    sel_msg = jnp.asarray(selm)
    seld = np.zeros((TW, HID), np.float32)
    for h in range(HEADS):
        seld[HID + h, h * DH:(h + 1) * DH] = 1.0
    sel_den = jnp.asarray(seld)

    zeros_acc = jnp.zeros((ACC_ROWS, TW), f32)

    si_vc = _pad_idx(edge_index_vc[0])
    di_vc = _pad_idx(edge_index_vc[1])
    si_cv = _pad_idx(edge_index_cv[0])
    di_cv = _pad_idx(edge_index_cv[1])

    h_c = _tc_linear(x_constraint, params["lin_dict"]["constraint"]["W"],
                     params["lin_dict"]["constraint"]["b"], "relu")
    h_v = _tc_linear(x_variable, params["lin_dict"]["variable"]["W"],
                     params["lin_dict"]["variable"]["b"], "relu")

    vc = "variable__to__constraint"
    cv = "constraint__to__variable"

    for p in params["convs"]:
        A_cv = _blockdiag(p["a_rel"][cv])
        M_cv = _blockdiag(p["m_rel"][cv])
        A_vc = _blockdiag(p["a_rel"][vc])
        M_vc = _blockdiag(p["m_rel"][vc])

        qt_c, kvt_c = _tc_qkv(
            h_c,
            p["q_lin"]["constraint"]["W"], p["q_lin"]["constraint"]["b"],
            p["k_lin"]["constraint"]["W"] @ A_cv, p["k_lin"]["constraint"]["b"] @ A_cv,
            p["v_lin"]["constraint"]["W"] @ M_cv, p["v_lin"]["constraint"]["b"] @ M_cv)
        qt_v, kvt_v = _tc_qkv(
            h_v,
            p["q_lin"]["variable"]["W"], p["q_lin"]["variable"]["b"],
            p["k_lin"]["variable"]["W"] @ A_vc, p["k_lin"]["variable"]["b"] @ A_vc,
            p["v_lin"]["variable"]["W"] @ M_vc, p["v_lin"]["variable"]["b"] @ M_vc)

        new_h = {}
        for ek, qt_d, kvt_s, si, di, dst_prev, dst in (
                (vc, qt_c, kvt_v, si_vc, di_vc, h_c, "constraint"),
                (cv, qt_v, kvt_c, si_cv, di_cv, h_v, "variable")):
            qg, kvg = _sc_gather3(qt_d, kvt_s, di, si)
            p_scaled = (p["p_rel"][ek] / np.sqrt(DH)).reshape(1, HEADS)
            logits, gmax = _tc_logits(qg, kvg, head_sum_pad, p_scaled)
            msg_ext = _tc_msg(logits, gmax, kvg, head_rep)
            parts = _sc_scatter(msg_ext, di, zeros_acc)
            x_prev_pad = jnp.pad(dst_prev, ((0, SEG_PAD - N_NODES), (0, 0)))
            new_h[dst] = _tc_combine(parts, x_prev_pad,
                                     p["a_lin"][dst]["W"], p["a_lin"][dst]["b"],
                                     p["skip"][dst], sel_msg, sel_den)[:N_NODES]
        h_c = new_h["constraint"]
        h_v = new_h["variable"]

    pooled = _tc_pool(h_c, h_v,
                      batch_constraint.reshape(-1, 1),
                      batch_variable.reshape(-1, 1),
                      params["lin"]["W"], params["lin"]["b"])
    return pooled[:, :2]


# speedup vs baseline: 1.0852x; 1.0852x over previous
import functools

import jax
import jax.numpy as jnp
import numpy as np
from jax import lax
from jax.experimental import pallas as pl
from jax.experimental.pallas import tpu as pltpu
from jax.experimental.pallas import tpu_sc as plsc

N_NODES = 25000
N_EDGES = 400000
IN_DIM = 128
HID = 64
HEADS = 4
DH = 16
NGRAPH = 16

SC_CORES = 2
SC_SUBCORES = 16
NW = SC_CORES * SC_SUBCORES

EDGE_PAD = 409600
PER_W = EDGE_PAD // NW
CHUNK = 128
NCHUNK = PER_W // CHUNK
SEG_PAD = 25088
ROWS_PER_TILE = SEG_PAD // SC_SUBCORES
TW = 128
SEG_HALF = SEG_PAD // 2
HALF_PER_TILE = SEG_HALF // SC_SUBCORES
ACC_ROWS = 12672
ZROWS_PER_TILE = ACC_ROWS // SC_SUBCORES
SCHUNK = 80
NCH_SC = EDGE_PAD // (SC_SUBCORES * SCHUNK)


def _tc_linear(x, W, b, act):
    M, K = x.shape
    N = W.shape[1]
    BM = 1000

    def body(x_ref, w_ref, b_ref, o_ref):
        acc = jnp.dot(x_ref[...], w_ref[...], preferred_element_type=jnp.float32)
        acc = acc + b_ref[...]
        if act == "relu":
            acc = jnp.maximum(acc, 0.0)
        o_ref[...] = acc

    return pl.pallas_call(
        body,
        grid=(M // BM,),
        in_specs=[
            pl.BlockSpec((BM, K), lambda i: (i, 0)),
            pl.BlockSpec((K, N), lambda i: (0, 0)),
            pl.BlockSpec((1, N), lambda i: (0, 0)),
        ],
        out_specs=pl.BlockSpec((BM, N), lambda i: (i, 0)),
        out_shape=jax.ShapeDtypeStruct((M, N), jnp.float32),
    )(x, W, b.reshape(1, -1))


def _tc_qkv(x, Wq, bq, Wk, bk, Wv, bv):
    M, K = x.shape
    BM = 1000

    def body(x_ref, wq_ref, bq_ref, wk_ref, bk_ref, wv_ref, bv_ref,
             qt_ref, kvt_ref):
        xb = x_ref[...]
        q = jnp.dot(xb, wq_ref[...], preferred_element_type=jnp.float32) + bq_ref[...]
        k = jnp.dot(xb, wk_ref[...], preferred_element_type=jnp.float32) + bk_ref[...]
        v = jnp.dot(xb, wv_ref[...], preferred_element_type=jnp.float32) + bv_ref[...]
        qt_ref[...] = jnp.concatenate([q, jnp.zeros((BM, HID), jnp.float32)], axis=1)
        kvt_ref[...] = jnp.concatenate([k, v], axis=1)

    wspec = pl.BlockSpec((K, HID), lambda i: (0, 0))
    bspec = pl.BlockSpec((1, HID), lambda i: (0, 0))
    ospec = pl.BlockSpec((BM, TW), lambda i: (i, 0))
    return pl.pallas_call(
        body,
        grid=(M // BM,),
        in_specs=[pl.BlockSpec((BM, K), lambda i: (i, 0)),
                  wspec, bspec, wspec, bspec, wspec, bspec],
        out_specs=[ospec, ospec],
        out_shape=[jax.ShapeDtypeStruct((M, TW), jnp.float32)] * 2,
    )(x, Wq, bq.reshape(1, -1), Wk, bk.reshape(1, -1), Wv, bv.reshape(1, -1))


def _tc_logits(mout, head_sum_pad, p_scaled):
    BE = 2048
    grid = EDGE_PAD // BE

    def body(m_ref, s_ref, p_ref, lg_ref, gm_ref):
        lg = jnp.dot(m_ref[...], s_ref[...],
                     preferred_element_type=jnp.float32) * p_ref[...]
        lg_ref[...] = lg
        bm = jnp.max(lg, axis=0, keepdims=True)

        @pl.when(pl.program_id(0) == 0)
        def _():
            gm_ref[...] = jnp.full((1, HEADS), -jnp.inf, jnp.float32)

        gm_ref[...] = jnp.maximum(gm_ref[...], bm)

    return pl.pallas_call(
        body,
        grid=(grid,),
        in_specs=[
            pl.BlockSpec((BE, TW), lambda i: (i, 0)),
            pl.BlockSpec((TW, HEADS), lambda i: (0, 0)),
            pl.BlockSpec((1, HEADS), lambda i: (0, 0)),
        ],
        out_specs=[pl.BlockSpec((BE, HEADS), lambda i: (i, 0)),
                   pl.BlockSpec((1, HEADS), lambda i: (0, 0))],
        out_shape=[jax.ShapeDtypeStruct((EDGE_PAD, HEADS), jnp.float32),
                   jax.ShapeDtypeStruct((1, HEADS), jnp.float32)],
    )(mout, head_sum_pad, p_scaled)


def _tc_msg(logits, gmax, mout, head_rep):
    BE = 2048
    grid = EDGE_PAD // BE

    def body(lg_ref, gm_ref, m_ref, r_ref, o_ref):
        ex = jnp.exp(lg_ref[...] - gm_ref[...])
        rows = pl.program_id(0) * BE + lax.broadcasted_iota(jnp.int32, (BE, HEADS), 0)
        ex = jnp.where(rows < N_EDGES, ex, 0.0)
        exb = jnp.dot(ex, r_ref[...], preferred_element_type=jnp.float32)
        msg = m_ref[...][:, HID:] * exb
        o_ref[...] = jnp.concatenate(
            [msg, ex, jnp.zeros((BE, TW - HID - HEADS), jnp.float32)], axis=1)

    return pl.pallas_call(
        body,
        grid=(grid,),
        in_specs=[
            pl.BlockSpec((BE, HEADS), lambda i: (i, 0)),
            pl.BlockSpec((1, HEADS), lambda i: (0, 0)),
            pl.BlockSpec((BE, TW), lambda i: (i, 0)),
            pl.BlockSpec((HEADS, HID), lambda i: (0, 0)),
        ],
        out_specs=pl.BlockSpec((BE, TW), lambda i: (i, 0)),
        out_shape=jax.ShapeDtypeStruct((EDGE_PAD, TW), jnp.float32),
    )(logits, gmax, mout, head_rep)


def _tc_combine(parts, x_prev_pad, Wa, ba, skip, sel_msg, sel_den):
    BM = SEG_HALF // 16

    def body(p_ref, x_ref, wa_ref, ba_ref, sk_ref, pm_ref, pd_ref, o_ref):
        s = p_ref[0]
        num = jnp.dot(s, pm_ref[...], preferred_element_type=jnp.float32)
        den = jnp.dot(s, pd_ref[...], preferred_element_type=jnp.float32)
        attn = num / (den + 1e-16)
        g = jax.nn.gelu(attn)
        o = jnp.dot(g, wa_ref[...], preferred_element_type=jnp.float32) + ba_ref[...]
        a = jax.nn.sigmoid(sk_ref[0, 0])
        o_ref[...] = a * o + (1.0 - a) * x_ref[...]

    return pl.pallas_call(
        body,
        grid=(SC_CORES, 16),
        in_specs=[
            pl.BlockSpec((1, BM, TW), lambda c, i: (c, i, 0)),
            pl.BlockSpec((BM, HID), lambda c, i: (c * 16 + i, 0)),
            pl.BlockSpec((HID, HID), lambda c, i: (0, 0)),
            pl.BlockSpec((1, HID), lambda c, i: (0, 0)),
            pl.BlockSpec((1, 1), lambda c, i: (0, 0)),
            pl.BlockSpec((TW, HID), lambda c, i: (0, 0)),
            pl.BlockSpec((TW, HID), lambda c, i: (0, 0)),
        ],
        out_specs=pl.BlockSpec((BM, HID), lambda c, i: (c * 16 + i, 0)),
        out_shape=jax.ShapeDtypeStruct((SEG_PAD, HID), jnp.float32),
    )(parts, x_prev_pad, Wa, ba.reshape(1, -1), skip.reshape(1, 1),
      sel_msg, sel_den)


def _tc_pool(xc, xv, bc, bv, Wl, bl):
    BM = 1000
    grid = N_NODES // BM

    def body(xc_ref, xv_ref, bc_ref, bv_ref, wl_ref, bl_ref, o_ref, accc, accv):
        pid = pl.program_id(0)

        @pl.when(pid == 0)
        def _():
            accc[...] = jnp.full((NGRAPH, HID), -jnp.inf, jnp.float32)
            accv[...] = jnp.full((NGRAPH, HID), -jnp.inf, jnp.float32)

        xcb = xc_ref[...]
        xvb = xv_ref[...]
        bcb = bc_ref[...]
        bvb = bv_ref[...]
        for g in range(NGRAPH):
            mc = jnp.where(bcb == g, xcb, -jnp.inf)
            mv = jnp.where(bvb == g, xvb, -jnp.inf)
            accc[g:g + 1, :] = jnp.maximum(accc[g:g + 1, :],
                                           jnp.max(mc, axis=0, keepdims=True))
            accv[g:g + 1, :] = jnp.maximum(accv[g:g + 1, :],
                                           jnp.max(mv, axis=0, keepdims=True))

        @pl.when(pid == grid - 1)
        def _():
            cp = accc[...]
            vp = accv[...]
            cp = jnp.where(jnp.isfinite(cp), cp, 0.0)
            vp = jnp.where(jnp.isfinite(vp), vp, 0.0)
            feat = jnp.concatenate([vp, cp], axis=1)
            lg = jnp.dot(feat, wl_ref[...], preferred_element_type=jnp.float32) + bl_ref[...]
            m = jnp.max(lg, axis=1, keepdims=True)
            e = jnp.exp(lg - m)
            p = e / jnp.sum(e, axis=1, keepdims=True)
            o_ref[...] = jnp.concatenate(
                [p, jnp.zeros((NGRAPH, 128 - 2), jnp.float32)], axis=1)

    return pl.pallas_call(
        body,
        grid=(grid,),
        in_specs=[
            pl.BlockSpec((BM, HID), lambda i: (i, 0)),
            pl.BlockSpec((BM, HID), lambda i: (i, 0)),
            pl.BlockSpec((BM, 1), lambda i: (i, 0)),
            pl.BlockSpec((BM, 1), lambda i: (i, 0)),
            pl.BlockSpec((2 * HID, 2), lambda i: (0, 0)),
            pl.BlockSpec((1, 2), lambda i: (0, 0)),
        ],
        out_specs=pl.BlockSpec((NGRAPH, 128), lambda i: (0, 0)),
        out_shape=jax.ShapeDtypeStruct((NGRAPH, 128), jnp.float32),
        scratch_shapes=[pltpu.VMEM((NGRAPH, HID), jnp.float32),
                        pltpu.VMEM((NGRAPH, HID), jnp.float32)],
    )(xc, xv, bc, bv, Wl, bl.reshape(1, -1))


_sc_cache = {}


def _sc_mesh():
    return plsc.VectorSubcoreMesh(core_axis_name="c", subcore_axis_name="s",
                                  num_cores=SC_CORES,
                                  num_subcores=SC_SUBCORES)


def _sc_gather3(qt, kvt, di, si):
    if "gather3" not in _sc_cache:
        @functools.partial(
            pl.kernel,
            out_type=jax.ShapeDtypeStruct((EDGE_PAD, TW), jnp.float32),
            mesh=_sc_mesh(),
            scratch_types=[
                pltpu.VMEM((PER_W,), jnp.int32),
                pltpu.VMEM((PER_W,), jnp.int32),
                [pltpu.VMEM((CHUNK, TW), jnp.float32) for _ in range(2)],
                [pltpu.VMEM((CHUNK, TW), jnp.float32) for _ in range(2)],
                [pltpu.SemaphoreType.DMA for _ in range(2)],
                [pltpu.SemaphoreType.DMA for _ in range(2)],
            ],
        )
        def body(qt_h, kvt_h, di_h, si_h, mout_h, di_all, si_all, qr, kr,
                 gsem, osem):
            wid = lax.axis_index("s") * SC_CORES + lax.axis_index("c")
            base0 = wid * PER_W

            pltpu.sync_copy(di_h.at[pl.ds(base0, PER_W)], di_all)
            pltpu.sync_copy(si_h.at[pl.ds(base0, PER_W)], si_all)

            def issue(cur, b):
                off = cur * CHUNK
                pltpu.async_copy(
                    qt_h.at[di_all.at[pl.ds(off, CHUNK)]], qr[b], gsem[b])
                pltpu.async_copy(
                    kvt_h.at[si_all.at[pl.ds(off, CHUNK)]], kr[b], gsem[b])

            def process(cur, b):
                off = cur * CHUNK
                pltpu.make_async_copy(
                    qt_h.at[di_all.at[pl.ds(off, CHUNK)]], qr[b],
                    gsem[b]).wait()
                pltpu.make_async_copy(
                    kvt_h.at[si_all.at[pl.ds(off, CHUNK)]], kr[b],
                    gsem[b]).wait()

                def row(r4, carry2):
                    for u in range(4):
                        for j in range(HID // 16):
                            sl = pl.ds(j * 16, 16)
                            kr[b][r4 * 4 + u, sl] = (kr[b][r4 * 4 + u, sl]
                                                     * qr[b][r4 * 4 + u, sl])
                    return carry2

                lax.fori_loop(0, CHUNK // 4, row, 0)
                base = base0 + cur * CHUNK
                return pltpu.async_copy(kr[b], mout_h.at[pl.ds(base, CHUNK)],
                                        osem[b])

            issue(0, 0)
            issue(1, 1)

            def pair(i, carry):
                g = i * 2
                for b in range(2):
                    cur = g + b
                    w = process(cur, b)
                    w.wait()
                    issue(cur + 2, b)
                return carry

            lax.fori_loop(0, (NCHUNK - 2) // 2, pair, 0)
            process(NCHUNK - 2, 0).wait()
            process(NCHUNK - 1, 1).wait()

        _sc_cache["gather3"] = body
    return _sc_cache["gather3"](qt, kvt, di, si)


def _sc_scatter(msg_ext, di, zeros_acc):
    if "scatter" not in _sc_cache:
        @functools.partial(
            pl.kernel,
            out_type=jax.ShapeDtypeStruct((SC_CORES * SEG_HALF, TW), jnp.float32),
            mesh=_sc_mesh(),
            scratch_types=[
                [pltpu.VMEM((SCHUNK,), jnp.int32) for _ in range(2)],
                [pltpu.VMEM((SCHUNK,), jnp.int32) for _ in range(2)],
                [pltpu.VMEM((SCHUNK, TW), jnp.float32) for _ in range(2)],
                pltpu.VMEM_SHARED((ACC_ROWS, TW), jnp.float32),
                [pltpu.SemaphoreType.DMA for _ in range(2)],
                [pltpu.SemaphoreType.DMA for _ in range(2)],
            ],
        )
        def body(msg_h, di_h, zeros_h, out_h, di_v, dloc_v, msg_v, acc_sh,
                 rsem, ssem):
            cid = lax.axis_index("c")
            sid = lax.axis_index("s")
            zrow0 = sid * ZROWS_PER_TILE
            lo = cid * SEG_HALF
            base0 = sid * NCH_SC * SCHUNK

            pltpu.sync_copy(zeros_h.at[pl.ds(zrow0, ZROWS_PER_TILE)],
                            acc_sh.at[pl.ds(zrow0, ZROWS_PER_TILE)])
            plsc.subcore_barrier()

            def issue(cur, b):
                base = base0 + cur * SCHUNK
                pltpu.async_copy(di_h.at[pl.ds(base, SCHUNK)], di_v[b], rsem[b])
                pltpu.async_copy(msg_h.at[pl.ds(base, SCHUNK)], msg_v[b], rsem[b])

            def process(cur, b):
                base = base0 + cur * SCHUNK
                pltpu.make_async_copy(di_h.at[pl.ds(base, SCHUNK)], di_v[b],
                                      rsem[b]).wait()
                pltpu.make_async_copy(msg_h.at[pl.ds(base, SCHUNK)], msg_v[b],
                                      rsem[b]).wait()
                for j in range(SCHUNK // 16):
                    sl = pl.ds(j * 16, 16)
                    d = di_v[b][sl] - lo
                    ok = (d >= 0) & (d < SEG_HALF)
                    dloc_v[b][sl] = jnp.where(ok, d, SEG_HALF)
                return pltpu.async_copy(msg_v[b], acc_sh.at[dloc_v[b]],
                                        ssem[b], add=True)

            issue(0, 0)
            issue(1, 1)

            def pair(i, carry):
                g = i * 2
                for b in range(2):
                    cur = g + b
                    w = process(cur, b)
                    w.wait()
                    issue(cur + 2, b)
                return carry

            lax.fori_loop(0, (NCH_SC - 2) // 2, pair, 0)
            process(NCH_SC - 2, 0).wait()
            process(NCH_SC - 1, 1).wait()
            plsc.subcore_barrier()

            row0 = sid * HALF_PER_TILE
            pltpu.sync_copy(acc_sh.at[pl.ds(row0, HALF_PER_TILE)],
                            out_h.at[pl.ds(cid * SEG_HALF + row0, HALF_PER_TILE)])

        _sc_cache["scatter"] = body
    return _sc_cache["scatter"](msg_ext, di, zeros_acc).reshape(
        SC_CORES, SEG_HALF, TW)


def _blockdiag(rel):
    return jax.scipy.linalg.block_diag(*[rel[h] for h in range(HEADS)])


def _pad_idx(idx):
    return jnp.concatenate([idx, jnp.zeros((EDGE_PAD - N_EDGES,), jnp.int32)])


def kernel(x_constraint, x_variable, edge_index_vc, edge_index_cv,
           batch_constraint, batch_variable, params):
    f32 = jnp.float32

    eye_h = np.zeros((HID, HEADS), np.float32)
    for h in range(HEADS):
        eye_h[h * DH:(h + 1) * DH, h] = 1.0
    head_sum_pad = jnp.asarray(
        np.concatenate([eye_h, np.zeros((TW - HID, HEADS), np.float32)]))
    head_rep = jnp.asarray(eye_h.T)
    selm = np.zeros((TW, HID), np.float32)
    selm[:HID, :HID] = np.eye(HID, dtype=np.float32)
    sel_msg = jnp.asarray(selm)
    seld = np.zeros((TW, HID), np.float32)
    for h in range(HEADS):
        seld[HID + h, h * DH:(h + 1) * DH] = 1.0
    sel_den = jnp.asarray(seld)

    zeros_acc = jnp.zeros((ACC_ROWS, TW), f32)

    si_vc = _pad_idx(edge_index_vc[0])
    di_vc = _pad_idx(edge_index_vc[1])
    si_cv = _pad_idx(edge_index_cv[0])
    di_cv = _pad_idx(edge_index_cv[1])

    h_c = _tc_linear(x_constraint, params["lin_dict"]["constraint"]["W"],
                     params["lin_dict"]["constraint"]["b"], "relu")
    h_v = _tc_linear(x_variable, params["lin_dict"]["variable"]["W"],
                     params["lin_dict"]["variable"]["b"], "relu")

    vc = "variable__to__constraint"
    cv = "constraint__to__variable"

    for p in params["convs"]:
        A_cv = _blockdiag(p["a_rel"][cv])
        M_cv = _blockdiag(p["m_rel"][cv])
        A_vc = _blockdiag(p["a_rel"][vc])
        M_vc = _blockdiag(p["m_rel"][vc])

        qt_c, kvt_c = _tc_qkv(
            h_c,
            p["q_lin"]["constraint"]["W"], p["q_lin"]["constraint"]["b"],
            p["k_lin"]["constraint"]["W"] @ A_cv, p["k_lin"]["constraint"]["b"] @ A_cv,
            p["v_lin"]["constraint"]["W"] @ M_cv, p["v_lin"]["constraint"]["b"] @ M_cv)
        qt_v, kvt_v = _tc_qkv(
            h_v,
            p["q_lin"]["variable"]["W"], p["q_lin"]["variable"]["b"],
            p["k_lin"]["variable"]["W"] @ A_vc, p["k_lin"]["variable"]["b"] @ A_vc,
            p["v_lin"]["variable"]["W"] @ M_vc, p["v_lin"]["variable"]["b"] @ M_vc)

        new_h = {}
        for ek, qt_d, kvt_s, si, di, dst_prev, dst in (
                (vc, qt_c, kvt_v, si_vc, di_vc, h_c, "constraint"),
                (cv, qt_v, kvt_c, si_cv, di_cv, h_v, "variable")):
            mout = _sc_gather3(qt_d, kvt_s, di, si)
            p_scaled = (p["p_rel"][ek] / np.sqrt(DH)).reshape(1, HEADS)
            logits, gmax = _tc_logits(mout, head_sum_pad, p_scaled)
            msg_ext = _tc_msg(logits, gmax, mout, head_rep)
            parts = _sc_scatter(msg_ext, di, zeros_acc)
            x_prev_pad = jnp.pad(dst_prev, ((0, SEG_PAD - N_NODES), (0, 0)))
            new_h[dst] = _tc_combine(parts, x_prev_pad,
                                     p["a_lin"][dst]["W"], p["a_lin"][dst]["b"],
                                     p["skip"][dst], sel_msg, sel_den)[:N_NODES]
        h_c = new_h["constraint"]
        h_v = new_h["variable"]

    pooled = _tc_pool(h_c, h_v,
                      batch_constraint.reshape(-1, 1),
                      batch_variable.reshape(-1, 1),
                      params["lin"]["W"], params["lin"]["b"])
    return pooled[:, :2]


# speedup vs baseline: 1.0854x; 1.0001x over previous
import functools

import jax
import jax.numpy as jnp
import numpy as np
from jax import lax
from jax.experimental import pallas as pl
from jax.experimental.pallas import tpu as pltpu
from jax.experimental.pallas import tpu_sc as plsc

N_NODES = 25000
N_EDGES = 400000
IN_DIM = 128
HID = 64
HEADS = 4
DH = 16
NGRAPH = 16

SC_CORES = 2
SC_SUBCORES = 16
NW = SC_CORES * SC_SUBCORES

EDGE_PAD = 409600
PER_W = EDGE_PAD // NW
CHUNK = 128
NCHUNK = PER_W // CHUNK
SEG_PAD = 25088
ROWS_PER_TILE = SEG_PAD // SC_SUBCORES
TW = 128
SEG_HALF = SEG_PAD // 2
HALF_PER_TILE = SEG_HALF // SC_SUBCORES
ACC_ROWS = 12672
ZROWS_PER_TILE = ACC_ROWS // SC_SUBCORES
SCHUNK = 80
NCH_SC = EDGE_PAD // (SC_SUBCORES * SCHUNK)


def _tc_linear(x, W, b, act):
    M, K = x.shape
    N = W.shape[1]
    BM = 1000

    def body(x_ref, w_ref, b_ref, o_ref):
        acc = jnp.dot(x_ref[...], w_ref[...], preferred_element_type=jnp.float32)
        acc = acc + b_ref[...]
        if act == "relu":
            acc = jnp.maximum(acc, 0.0)
        o_ref[...] = acc

    return pl.pallas_call(
        body,
        grid=(M // BM,),
        in_specs=[
            pl.BlockSpec((BM, K), lambda i: (i, 0)),
            pl.BlockSpec((K, N), lambda i: (0, 0)),
            pl.BlockSpec((1, N), lambda i: (0, 0)),
        ],
        out_specs=pl.BlockSpec((BM, N), lambda i: (i, 0)),
        out_shape=jax.ShapeDtypeStruct((M, N), jnp.float32),
    )(x, W, b.reshape(1, -1))


def _tc_qkv(x, Wq, bq, Wk, bk, Wv, bv):
    M, K = x.shape
    BM = 1000

    def body(x_ref, wq_ref, bq_ref, wk_ref, bk_ref, wv_ref, bv_ref,
             qt_ref, kvt_ref):
        xb = x_ref[...]
        q = jnp.dot(xb, wq_ref[...], preferred_element_type=jnp.float32) + bq_ref[...]
        k = jnp.dot(xb, wk_ref[...], preferred_element_type=jnp.float32) + bk_ref[...]
        v = jnp.dot(xb, wv_ref[...], preferred_element_type=jnp.float32) + bv_ref[...]
        qt_ref[...] = jnp.concatenate([q, jnp.zeros((BM, HID), jnp.float32)], axis=1)
        kvt_ref[...] = jnp.concatenate([k, v], axis=1)

    wspec = pl.BlockSpec((K, HID), lambda i: (0, 0))
    bspec = pl.BlockSpec((1, HID), lambda i: (0, 0))
    ospec = pl.BlockSpec((BM, TW), lambda i: (i, 0))
    return pl.pallas_call(
        body,
        grid=(M // BM,),
        in_specs=[pl.BlockSpec((BM, K), lambda i: (i, 0)),
                  wspec, bspec, wspec, bspec, wspec, bspec],
        out_specs=[ospec, ospec],
        out_shape=[jax.ShapeDtypeStruct((M, TW), jnp.float32)] * 2,
    )(x, Wq, bq.reshape(1, -1), Wk, bk.reshape(1, -1), Wv, bv.reshape(1, -1))


def _tc_logits(mout, head_sum_pad, p_scaled):
    BE = 2048
    grid = EDGE_PAD // BE

    def body(m_ref, s_ref, p_ref, lg_ref, gm_ref):
        lg = jnp.dot(m_ref[...], s_ref[...],
                     preferred_element_type=jnp.float32) * p_ref[...]
        lg_ref[...] = lg
        bm = jnp.max(lg, axis=0, keepdims=True)

        @pl.when(pl.program_id(0) == 0)
        def _():
            gm_ref[...] = jnp.full((1, HEADS), -jnp.inf, jnp.float32)

        gm_ref[...] = jnp.maximum(gm_ref[...], bm)

    return pl.pallas_call(
        body,
        grid=(grid,),
        in_specs=[
            pl.BlockSpec((BE, TW), lambda i: (i, 0)),
            pl.BlockSpec((TW, HEADS), lambda i: (0, 0)),
            pl.BlockSpec((1, HEADS), lambda i: (0, 0)),
        ],
        out_specs=[pl.BlockSpec((BE, HEADS), lambda i: (i, 0)),
                   pl.BlockSpec((1, HEADS), lambda i: (0, 0))],
        out_shape=[jax.ShapeDtypeStruct((EDGE_PAD, HEADS), jnp.float32),
                   jax.ShapeDtypeStruct((1, HEADS), jnp.float32)],
    )(mout, head_sum_pad, p_scaled)


def _tc_msg(logits, gmax, mout, head_rep):
    BE = 2048
    grid = EDGE_PAD // BE

    def body(lg_ref, gm_ref, m_ref, r_ref, o_ref):
        ex = jnp.exp(lg_ref[...] - gm_ref[...])
        rows = pl.program_id(0) * BE + lax.broadcasted_iota(jnp.int32, (BE, HEADS), 0)
        ex = jnp.where(rows < N_EDGES, ex, 0.0)
        exb = jnp.dot(ex, r_ref[...], preferred_element_type=jnp.float32)
        msg = m_ref[...][:, HID:] * exb
        o_ref[...] = jnp.concatenate(
            [msg, ex, jnp.zeros((BE, TW - HID - HEADS), jnp.float32)], axis=1)

    return pl.pallas_call(
        body,
        grid=(grid,),
        in_specs=[
            pl.BlockSpec((BE, HEADS), lambda i: (i, 0)),
            pl.BlockSpec((1, HEADS), lambda i: (0, 0)),
            pl.BlockSpec((BE, TW), lambda i: (i, 0)),
            pl.BlockSpec((HEADS, HID), lambda i: (0, 0)),
        ],
        out_specs=pl.BlockSpec((BE, TW), lambda i: (i, 0)),
        out_shape=jax.ShapeDtypeStruct((EDGE_PAD, TW), jnp.float32),
    )(logits, gmax, mout, head_rep)


def _tc_combine(parts, x_prev_pad, Wa, ba, skip, sel_msg, sel_den):
    BM = SEG_HALF // 16

    def body(p_ref, x_ref, wa_ref, ba_ref, sk_ref, pm_ref, pd_ref, o_ref):
        s = p_ref[0]
        num = jnp.dot(s, pm_ref[...], preferred_element_type=jnp.float32)
        den = jnp.dot(s, pd_ref[...], preferred_element_type=jnp.float32)
        attn = num / (den + 1e-16)
        g = jax.nn.gelu(attn)
        o = jnp.dot(g, wa_ref[...], preferred_element_type=jnp.float32) + ba_ref[...]
        a = jax.nn.sigmoid(sk_ref[0, 0])
        o_ref[...] = a * o + (1.0 - a) * x_ref[...]

    return pl.pallas_call(
        body,
        grid=(SC_CORES, 16),
        in_specs=[
            pl.BlockSpec((1, BM, TW), lambda c, i: (c, i, 0)),
            pl.BlockSpec((BM, HID), lambda c, i: (c * 16 + i, 0)),
            pl.BlockSpec((HID, HID), lambda c, i: (0, 0)),
            pl.BlockSpec((1, HID), lambda c, i: (0, 0)),
            pl.BlockSpec((1, 1), lambda c, i: (0, 0)),
            pl.BlockSpec((TW, HID), lambda c, i: (0, 0)),
            pl.BlockSpec((TW, HID), lambda c, i: (0, 0)),
        ],
        out_specs=pl.BlockSpec((BM, HID), lambda c, i: (c * 16 + i, 0)),
        out_shape=jax.ShapeDtypeStruct((SEG_PAD, HID), jnp.float32),
    )(parts, x_prev_pad, Wa, ba.reshape(1, -1), skip.reshape(1, 1),
      sel_msg, sel_den)


def _tc_pool(xc, xv, bc, bv, Wl, bl):
    BM = 1000
    grid = N_NODES // BM

    def body(xc_ref, xv_ref, bc_ref, bv_ref, wl_ref, bl_ref, o_ref, accc, accv):
        pid = pl.program_id(0)

        @pl.when(pid == 0)
        def _():
            accc[...] = jnp.full((NGRAPH, HID), -jnp.inf, jnp.float32)
            accv[...] = jnp.full((NGRAPH, HID), -jnp.inf, jnp.float32)

        xcb = xc_ref[...]
        xvb = xv_ref[...]
        bcb = bc_ref[...]
        bvb = bv_ref[...]
        for g in range(NGRAPH):
            mc = jnp.where(bcb == g, xcb, -jnp.inf)
            mv = jnp.where(bvb == g, xvb, -jnp.inf)
            accc[g:g + 1, :] = jnp.maximum(accc[g:g + 1, :],
                                           jnp.max(mc, axis=0, keepdims=True))
            accv[g:g + 1, :] = jnp.maximum(accv[g:g + 1, :],
                                           jnp.max(mv, axis=0, keepdims=True))

        @pl.when(pid == grid - 1)
        def _():
            cp = accc[...]
            vp = accv[...]
            cp = jnp.where(jnp.isfinite(cp), cp, 0.0)
            vp = jnp.where(jnp.isfinite(vp), vp, 0.0)
            feat = jnp.concatenate([vp, cp], axis=1)
            lg = jnp.dot(feat, wl_ref[...], preferred_element_type=jnp.float32) + bl_ref[...]
            m = jnp.max(lg, axis=1, keepdims=True)
            e = jnp.exp(lg - m)
            p = e / jnp.sum(e, axis=1, keepdims=True)
            o_ref[...] = jnp.concatenate(
                [p, jnp.zeros((NGRAPH, 128 - 2), jnp.float32)], axis=1)

    return pl.pallas_call(
        body,
        grid=(grid,),
        in_specs=[
            pl.BlockSpec((BM, HID), lambda i: (i, 0)),
            pl.BlockSpec((BM, HID), lambda i: (i, 0)),
            pl.BlockSpec((BM, 1), lambda i: (i, 0)),
            pl.BlockSpec((BM, 1), lambda i: (i, 0)),
            pl.BlockSpec((2 * HID, 2), lambda i: (0, 0)),
            pl.BlockSpec((1, 2), lambda i: (0, 0)),
        ],
        out_specs=pl.BlockSpec((NGRAPH, 128), lambda i: (0, 0)),
        out_shape=jax.ShapeDtypeStruct((NGRAPH, 128), jnp.float32),
        scratch_shapes=[pltpu.VMEM((NGRAPH, HID), jnp.float32),
                        pltpu.VMEM((NGRAPH, HID), jnp.float32)],
    )(xc, xv, bc, bv, Wl, bl.reshape(1, -1))


_sc_cache = {}


def _sc_mesh():
    return plsc.VectorSubcoreMesh(core_axis_name="c", subcore_axis_name="s",
                                  num_cores=SC_CORES,
                                  num_subcores=SC_SUBCORES)


def _sc_gather3(qt, kvt, di, si):
    if "gather3" not in _sc_cache:
        @functools.partial(
            pl.kernel,
            out_type=jax.ShapeDtypeStruct((EDGE_PAD, TW), jnp.float32),
            mesh=_sc_mesh(),
            scratch_types=[
                pltpu.VMEM((PER_W,), jnp.int32),
                pltpu.VMEM((PER_W,), jnp.int32),
                [pltpu.VMEM((CHUNK, TW), jnp.float32) for _ in range(2)],
                [pltpu.VMEM((CHUNK, TW), jnp.float32) for _ in range(2)],
                [pltpu.SemaphoreType.DMA for _ in range(2)],
                [pltpu.SemaphoreType.DMA for _ in range(2)],
            ],
        )
        def body(qt_h, kvt_h, di_h, si_h, mout_h, di_all, si_all, qr, kr,
                 gsem, osem):
            wid = lax.axis_index("s") * SC_CORES + lax.axis_index("c")
            base0 = wid * PER_W

            pltpu.sync_copy(di_h.at[pl.ds(base0, PER_W)], di_all)
            pltpu.sync_copy(si_h.at[pl.ds(base0, PER_W)], si_all)

            def issue(cur, b):
                off = cur * CHUNK
                pltpu.async_copy(
                    qt_h.at[di_all.at[pl.ds(off, CHUNK)]], qr[b], gsem[b])
                pltpu.async_copy(
                    kvt_h.at[si_all.at[pl.ds(off, CHUNK)]], kr[b], gsem[b])

            def process(cur, b):
                off = cur * CHUNK
                pltpu.make_async_copy(
                    qt_h.at[di_all.at[pl.ds(off, CHUNK)]], qr[b],
                    gsem[b]).wait()
                pltpu.make_async_copy(
                    kvt_h.at[si_all.at[pl.ds(off, CHUNK)]], kr[b],
                    gsem[b]).wait()

                def row(r4, carry2):
                    for u in range(4):
                        for j in range(HID // 16):
                            sl = pl.ds(j * 16, 16)
                            kr[b][r4 * 4 + u, sl] = (kr[b][r4 * 4 + u, sl]
                                                     * qr[b][r4 * 4 + u, sl])
                    return carry2

                lax.fori_loop(0, CHUNK // 4, row, 0)
                base = base0 + cur * CHUNK
                return pltpu.async_copy(kr[b], mout_h.at[pl.ds(base, CHUNK)],
                                        osem[b])

            issue(0, 0)
            issue(1, 1)

            def pair(i, carry):
                g = i * 2
                for b in range(2):
                    cur = g + b
                    w = process(cur, b)
                    w.wait()
                    issue(cur + 2, b)
                return carry

            lax.fori_loop(0, (NCHUNK - 2) // 2, pair, 0)
            process(NCHUNK - 2, 0).wait()
            process(NCHUNK - 1, 1).wait()

        _sc_cache["gather3"] = body
    return _sc_cache["gather3"](qt, kvt, di, si)


def _sc_scatter(msg_ext, di, zeros_acc):
    if "scatter" not in _sc_cache:
        @functools.partial(
            pl.kernel,
            out_type=jax.ShapeDtypeStruct((SC_CORES * SEG_HALF, TW), jnp.float32),
            mesh=_sc_mesh(),
            scratch_types=[
                [pltpu.VMEM((SCHUNK,), jnp.int32) for _ in range(2)],
                [pltpu.VMEM((SCHUNK,), jnp.int32) for _ in range(2)],
                [pltpu.VMEM((SCHUNK, TW), jnp.float32) for _ in range(2)],
                pltpu.VMEM_SHARED((ACC_ROWS, TW), jnp.float32),
                [pltpu.SemaphoreType.DMA for _ in range(2)],
                [pltpu.SemaphoreType.DMA for _ in range(2)],
            ],
        )
        def body(msg_h, di_h, zeros_h, out_h, di_v, dloc_v, msg_v, acc_sh,
                 rsem, ssem):
            cid = lax.axis_index("c")
            sid = lax.axis_index("s")
            zrow0 = sid * ZROWS_PER_TILE
            lo = cid * SEG_HALF
            base0 = sid * NCH_SC * SCHUNK

            pltpu.sync_copy(zeros_h.at[pl.ds(zrow0, ZROWS_PER_TILE)],
                            acc_sh.at[pl.ds(zrow0, ZROWS_PER_TILE)])
            plsc.subcore_barrier()

            def issue(cur, b):
                base = base0 + cur * SCHUNK
                pltpu.async_copy(di_h.at[pl.ds(base, SCHUNK)], di_v[b], rsem[b])
                pltpu.async_copy(msg_h.at[pl.ds(base, SCHUNK)], msg_v[b], rsem[b])

            def process(cur, b):
                base = base0 + cur * SCHUNK
                pltpu.make_async_copy(di_h.at[pl.ds(base, SCHUNK)], di_v[b],
                                      rsem[b]).wait()
                pltpu.make_async_copy(msg_h.at[pl.ds(base, SCHUNK)], msg_v[b],
                                      rsem[b]).wait()
                for j in range(SCHUNK // 16):
                    sl = pl.ds(j * 16, 16)
                    d = di_v[b][sl] - lo
                    ok = (d >= 0) & (d < SEG_HALF)
                    dloc_v[b][sl] = jnp.where(ok, d, SEG_HALF)
                return pltpu.async_copy(msg_v[b], acc_sh.at[dloc_v[b]],
                                        ssem[b], add=True)

            issue(0, 0)
            issue(1, 1)

            def pair(i, carry):
                g = i * 2
                for b in range(2):
                    cur = g + b
                    w = process(cur, b)
                    w.wait()
                    issue(cur + 2, b)
                return carry

            lax.fori_loop(0, (NCH_SC - 2) // 2, pair, 0)
            process(NCH_SC - 2, 0).wait()
            process(NCH_SC - 1, 1).wait()
            plsc.subcore_barrier()

            row0 = sid * HALF_PER_TILE
            pltpu.sync_copy(acc_sh.at[pl.ds(row0, HALF_PER_TILE)],
                            out_h.at[pl.ds(cid * SEG_HALF + row0, HALF_PER_TILE)])

        _sc_cache["scatter"] = body
    return _sc_cache["scatter"](msg_ext, di, zeros_acc).reshape(
        SC_CORES, SEG_HALF, TW)


def _blockdiag(rel):
    return jax.scipy.linalg.block_diag(*[rel[h] for h in range(HEADS)])


def _pad_idx(idx):
    return jnp.concatenate([idx, jnp.zeros((EDGE_PAD - N_EDGES,), jnp.int32)])


def kernel(x_constraint, x_variable, edge_index_vc, edge_index_cv,
           batch_constraint, batch_variable, params):
    f32 = jnp.float32

    eye_h = np.zeros((HID, HEADS), np.float32)
    for h in range(HEADS):
        eye_h[h * DH:(h + 1) * DH, h] = 1.0
    head_sum_pad = jnp.asarray(
        np.concatenate([eye_h, np.zeros((TW - HID, HEADS), np.float32)]))
    head_rep = jnp.asarray(eye_h.T)
    selm = np.zeros((TW, HID), np.float32)
    selm[:HID, :HID] = np.eye(HID, dtype=np.float32)
    sel_msg = jnp.asarray(selm)
    seld = np.zeros((TW, HID), np.float32)
    for h in range(HEADS):
        seld[HID + h, h * DH:(h + 1) * DH] = 1.0
    sel_den = jnp.asarray(seld)

    zeros_acc = jnp.zeros((ACC_ROWS, TW), f32)

    si_vc = _pad_idx(edge_index_vc[0])
    di_vc = _pad_idx(edge_index_vc[1])
    si_cv = _pad_idx(edge_index_cv[0])
    di_cv = _pad_idx(edge_index_cv[1])

    h_c = _tc_linear(x_constraint, params["lin_dict"]["constraint"]["W"],
                     params["lin_dict"]["constraint"]["b"], "relu")
    h_v = _tc_linear(x_variable, params["lin_dict"]["variable"]["W"],
                     params["lin_dict"]["variable"]["b"], "relu")

    vc = "variable__to__constraint"
    cv = "constraint__to__variable"

    for p in params["convs"]:
        A_cv = _blockdiag(p["a_rel"][cv])
        M_cv = _blockdiag(p["m_rel"][cv])
        A_vc = _blockdiag(p["a_rel"][vc])
        M_vc = _blockdiag(p["m_rel"][vc])

        qt_c, kvt_c = _tc_qkv(
            h_c,
            p["q_lin"]["constraint"]["W"], p["q_lin"]["constraint"]["b"],
            p["k_lin"]["constraint"]["W"] @ A_cv, p["k_lin"]["constraint"]["b"] @ A_cv,
            p["v_lin"]["constraint"]["W"] @ M_cv, p["v_lin"]["constraint"]["b"] @ M_cv)
        qt_v, kvt_v = _tc_qkv(
            h_v,
            p["q_lin"]["variable"]["W"], p["q_lin"]["variable"]["b"],
            p["k_lin"]["variable"]["W"] @ A_vc, p["k_lin"]["variable"]["b"] @ A_vc,
            p["v_lin"]["variable"]["W"] @ M_vc, p["v_lin"]["variable"]["b"] @ M_vc)

        mout_vc = _sc_gather3(qt_c, kvt_v, di_vc, si_vc)
        mout_cv = _sc_gather3(qt_v, kvt_c, di_cv, si_cv)
        ps_vc = (p["p_rel"][vc] / np.sqrt(DH)).reshape(1, HEADS)
        ps_cv = (p["p_rel"][cv] / np.sqrt(DH)).reshape(1, HEADS)
        lg_vc, gm_vc = _tc_logits(mout_vc, head_sum_pad, ps_vc)
        msg_vc = _tc_msg(lg_vc, gm_vc, mout_vc, head_rep)
        parts_vc = _sc_scatter(msg_vc, di_vc, zeros_acc)
        lg_cv, gm_cv = _tc_logits(mout_cv, head_sum_pad, ps_cv)
        msg_cv = _tc_msg(lg_cv, gm_cv, mout_cv, head_rep)
        parts_cv = _sc_scatter(msg_cv, di_cv, zeros_acc)
        hc_pad = jnp.pad(h_c, ((0, SEG_PAD - N_NODES), (0, 0)))
        hv_pad = jnp.pad(h_v, ((0, SEG_PAD - N_NODES), (0, 0)))
        h_c = _tc_combine(parts_vc, hc_pad,
                          p["a_lin"]["constraint"]["W"],
                          p["a_lin"]["constraint"]["b"],
                          p["skip"]["constraint"], sel_msg, sel_den)[:N_NODES]
        h_v = _tc_combine(parts_cv, hv_pad,
                          p["a_lin"]["variable"]["W"],
                          p["a_lin"]["variable"]["b"],
                          p["skip"]["variable"], sel_msg, sel_den)[:N_NODES]

    pooled = _tc_pool(h_c, h_v,
                      batch_constraint.reshape(-1, 1),
                      batch_variable.reshape(-1, 1),
                      params["lin"]["W"], params["lin"]["b"])
    return pooled[:, :2]


# speedup vs baseline: 1.0873x; 1.0018x over previous
import functools

import jax
import jax.numpy as jnp
import numpy as np
from jax import lax
from jax.experimental import pallas as pl
from jax.experimental.pallas import tpu as pltpu
from jax.experimental.pallas import tpu_sc as plsc

N_NODES = 25000
N_EDGES = 400000
IN_DIM = 128
HID = 64
HEADS = 4
DH = 16
NGRAPH = 16

SC_CORES = 2
SC_SUBCORES = 16
NW = SC_CORES * SC_SUBCORES

EDGE_PAD = 409600
PER_W = EDGE_PAD // NW
CHUNK = 128
NCHUNK = PER_W // CHUNK
SEG_PAD = 25088
ROWS_PER_TILE = SEG_PAD // SC_SUBCORES
TW = 128
SEG_HALF = SEG_PAD // 2
HALF_PER_TILE = SEG_HALF // SC_SUBCORES
ACC_ROWS = 12672
ZROWS_PER_TILE = ACC_ROWS // SC_SUBCORES
SCHUNK = 80
NCH_SC = EDGE_PAD // (SC_SUBCORES * SCHUNK)


def _tc_linear(x, W, b, act):
    M, K = x.shape
    N = W.shape[1]
    BM = 1000

    def body(x_ref, w_ref, b_ref, o_ref):
        acc = jnp.dot(x_ref[...], w_ref[...], preferred_element_type=jnp.float32)
        acc = acc + b_ref[...]
        if act == "relu":
            acc = jnp.maximum(acc, 0.0)
        o_ref[...] = acc

    return pl.pallas_call(
        body,
        grid=(M // BM,),
        in_specs=[
            pl.BlockSpec((BM, K), lambda i: (i, 0)),
            pl.BlockSpec((K, N), lambda i: (0, 0)),
            pl.BlockSpec((1, N), lambda i: (0, 0)),
        ],
        out_specs=pl.BlockSpec((BM, N), lambda i: (i, 0)),
        out_shape=jax.ShapeDtypeStruct((M, N), jnp.float32),
    )(x, W, b.reshape(1, -1))


def _tc_qkv(x, Wq, bq, Wk, bk, Wv, bv):
    M, K = x.shape
    BM = 1000

    def body(x_ref, wq_ref, bq_ref, wk_ref, bk_ref, wv_ref, bv_ref,
             qt_ref, kvt_ref):
        xb = x_ref[...]
        q = jnp.dot(xb, wq_ref[...], preferred_element_type=jnp.float32) + bq_ref[...]
        k = jnp.dot(xb, wk_ref[...], preferred_element_type=jnp.float32) + bk_ref[...]
        v = jnp.dot(xb, wv_ref[...], preferred_element_type=jnp.float32) + bv_ref[...]
        qt_ref[...] = jnp.concatenate([q, jnp.zeros((BM, HID), jnp.float32)], axis=1)
        kvt_ref[...] = jnp.concatenate([k, v], axis=1)

    wspec = pl.BlockSpec((K, HID), lambda i: (0, 0))
    bspec = pl.BlockSpec((1, HID), lambda i: (0, 0))
    ospec = pl.BlockSpec((BM, TW), lambda i: (i, 0))
    return pl.pallas_call(
        body,
        grid=(M // BM,),
        in_specs=[pl.BlockSpec((BM, K), lambda i: (i, 0)),
                  wspec, bspec, wspec, bspec, wspec, bspec],
        out_specs=[ospec, ospec],
        out_shape=[jax.ShapeDtypeStruct((M, TW), jnp.float32)] * 2,
    )(x, Wq, bq.reshape(1, -1), Wk, bk.reshape(1, -1), Wv, bv.reshape(1, -1))


def _tc_logits(mout, head_sum_pad, p_scaled):
    BE = 2048
    grid = EDGE_PAD // BE

    def body(m_ref, s_ref, p_ref, lg_ref, gm_ref):
        lg = jnp.dot(m_ref[...], s_ref[...],
                     preferred_element_type=jnp.float32) * p_ref[...]
        lg_ref[...] = lg
        bm = jnp.max(lg, axis=0, keepdims=True)

        @pl.when(pl.program_id(0) == 0)
        def _():
            gm_ref[...] = jnp.full((1, HEADS), -jnp.inf, jnp.float32)

        gm_ref[...] = jnp.maximum(gm_ref[...], bm)

    return pl.pallas_call(
        body,
        grid=(grid,),
        in_specs=[
            pl.BlockSpec((BE, TW), lambda i: (i, 0)),
            pl.BlockSpec((TW, HEADS), lambda i: (0, 0)),
            pl.BlockSpec((1, HEADS), lambda i: (0, 0)),
        ],
        out_specs=[pl.BlockSpec((BE, HEADS), lambda i: (i, 0)),
                   pl.BlockSpec((1, HEADS), lambda i: (0, 0))],
        out_shape=[jax.ShapeDtypeStruct((EDGE_PAD, HEADS), jnp.float32),
                   jax.ShapeDtypeStruct((1, HEADS), jnp.float32)],
    )(mout, head_sum_pad, p_scaled)


def _tc_msg(logits, gmax, mout, head_rep):
    BE = 2048
    grid = EDGE_PAD // BE

    def body(lg_ref, gm_ref, m_ref, r_ref, o_ref):
        ex = jnp.exp(lg_ref[...] - gm_ref[...])
        rows = pl.program_id(0) * BE + lax.broadcasted_iota(jnp.int32, (BE, HEADS), 0)
        ex = jnp.where(rows < N_EDGES, ex, 0.0)
        exb = jnp.dot(ex, r_ref[...], preferred_element_type=jnp.float32)
        msg = m_ref[...][:, HID:] * exb
        o_ref[...] = jnp.concatenate(
            [msg, ex, jnp.zeros((BE, TW - HID - HEADS), jnp.float32)], axis=1)

    return pl.pallas_call(
        body,
        grid=(grid,),
        in_specs=[
            pl.BlockSpec((BE, HEADS), lambda i: (i, 0)),
            pl.BlockSpec((1, HEADS), lambda i: (0, 0)),
            pl.BlockSpec((BE, TW), lambda i: (i, 0)),
            pl.BlockSpec((HEADS, HID), lambda i: (0, 0)),
        ],
        out_specs=pl.BlockSpec((BE, TW), lambda i: (i, 0)),
        out_shape=jax.ShapeDtypeStruct((EDGE_PAD, TW), jnp.float32),
    )(logits, gmax, mout, head_rep)


def _tc_combine(parts, x_prev_pad, Wa, ba, skip, sel_msg, sel_den):
    BM = SEG_HALF // 16

    def body(p_ref, x_ref, wa_ref, ba_ref, sk_ref, pm_ref, pd_ref, o_ref):
        s = p_ref[0]
        num = jnp.dot(s, pm_ref[...], preferred_element_type=jnp.float32)
        den = jnp.dot(s, pd_ref[...], preferred_element_type=jnp.float32)
        attn = num / (den + 1e-16)
        g = jax.nn.gelu(attn)
        o = jnp.dot(g, wa_ref[...], preferred_element_type=jnp.float32) + ba_ref[...]
        a = jax.nn.sigmoid(sk_ref[0, 0])
        o_ref[...] = a * o + (1.0 - a) * x_ref[...]

    return pl.pallas_call(
        body,
        grid=(SC_CORES, 16),
        in_specs=[
            pl.BlockSpec((1, BM, TW), lambda c, i: (c, i, 0)),
            pl.BlockSpec((BM, HID), lambda c, i: (c * 16 + i, 0)),
            pl.BlockSpec((HID, HID), lambda c, i: (0, 0)),
            pl.BlockSpec((1, HID), lambda c, i: (0, 0)),
            pl.BlockSpec((1, 1), lambda c, i: (0, 0)),
            pl.BlockSpec((TW, HID), lambda c, i: (0, 0)),
            pl.BlockSpec((TW, HID), lambda c, i: (0, 0)),
        ],
        out_specs=pl.BlockSpec((BM, HID), lambda c, i: (c * 16 + i, 0)),
        out_shape=jax.ShapeDtypeStruct((SEG_PAD, HID), jnp.float32),
    )(parts, x_prev_pad, Wa, ba.reshape(1, -1), skip.reshape(1, 1),
      sel_msg, sel_den)


def _tc_pool(xc, xv, bc, bv, Wl, bl):
    BM = 1000
    grid = N_NODES // BM

    def body(xc_ref, xv_ref, bc_ref, bv_ref, wl_ref, bl_ref, o_ref, accc, accv):
        pid = pl.program_id(0)

        @pl.when(pid == 0)
        def _():
            accc[...] = jnp.full((NGRAPH, HID), -jnp.inf, jnp.float32)
            accv[...] = jnp.full((NGRAPH, HID), -jnp.inf, jnp.float32)

        xcb = xc_ref[...]
        xvb = xv_ref[...]
        bcb = bc_ref[...]
        bvb = bv_ref[...]
        for g in range(NGRAPH):
            mc = jnp.where(bcb == g, xcb, -jnp.inf)
            mv = jnp.where(bvb == g, xvb, -jnp.inf)
            accc[g:g + 1, :] = jnp.maximum(accc[g:g + 1, :],
                                           jnp.max(mc, axis=0, keepdims=True))
            accv[g:g + 1, :] = jnp.maximum(accv[g:g + 1, :],
                                           jnp.max(mv, axis=0, keepdims=True))

        @pl.when(pid == grid - 1)
        def _():
            cp = accc[...]
            vp = accv[...]
            cp = jnp.where(jnp.isfinite(cp), cp, 0.0)
            vp = jnp.where(jnp.isfinite(vp), vp, 0.0)
            feat = jnp.concatenate([vp, cp], axis=1)
            lg = jnp.dot(feat, wl_ref[...], preferred_element_type=jnp.float32) + bl_ref[...]
            m = jnp.max(lg, axis=1, keepdims=True)
            e = jnp.exp(lg - m)
            p = e / jnp.sum(e, axis=1, keepdims=True)
            o_ref[...] = jnp.concatenate(
                [p, jnp.zeros((NGRAPH, 128 - 2), jnp.float32)], axis=1)

    return pl.pallas_call(
        body,
        grid=(grid,),
        in_specs=[
            pl.BlockSpec((BM, HID), lambda i: (i, 0)),
            pl.BlockSpec((BM, HID), lambda i: (i, 0)),
            pl.BlockSpec((BM, 1), lambda i: (i, 0)),
            pl.BlockSpec((BM, 1), lambda i: (i, 0)),
            pl.BlockSpec((2 * HID, 2), lambda i: (0, 0)),
            pl.BlockSpec((1, 2), lambda i: (0, 0)),
        ],
        out_specs=pl.BlockSpec((NGRAPH, 128), lambda i: (0, 0)),
        out_shape=jax.ShapeDtypeStruct((NGRAPH, 128), jnp.float32),
        scratch_shapes=[pltpu.VMEM((NGRAPH, HID), jnp.float32),
                        pltpu.VMEM((NGRAPH, HID), jnp.float32)],
    )(xc, xv, bc, bv, Wl, bl.reshape(1, -1))


_sc_cache = {}


def _sc_mesh():
    return plsc.VectorSubcoreMesh(core_axis_name="c", subcore_axis_name="s",
                                  num_cores=SC_CORES,
                                  num_subcores=SC_SUBCORES)


def _sc_gather3(qt, kvt, di, si):
    if "gather3" not in _sc_cache:
        @functools.partial(
            pl.kernel,
            out_type=jax.ShapeDtypeStruct((EDGE_PAD, TW), jnp.float32),
            mesh=_sc_mesh(),
            scratch_types=[
                pltpu.VMEM((PER_W,), jnp.int32),
                pltpu.VMEM((PER_W,), jnp.int32),
                [pltpu.VMEM((CHUNK, TW), jnp.float32) for _ in range(3)],
                [pltpu.VMEM((CHUNK, TW), jnp.float32) for _ in range(3)],
                [pltpu.SemaphoreType.DMA for _ in range(3)],
                [pltpu.SemaphoreType.DMA for _ in range(3)],
            ],
        )
        def body(qt_h, kvt_h, di_h, si_h, mout_h, di_all, si_all, qr, kr,
                 gsem, osem):
            wid = lax.axis_index("s") * SC_CORES + lax.axis_index("c")
            base0 = wid * PER_W

            pltpu.sync_copy(di_h.at[pl.ds(base0, PER_W)], di_all)
            pltpu.sync_copy(si_h.at[pl.ds(base0, PER_W)], si_all)

            def issue(cur, b):
                off = cur * CHUNK
                pltpu.async_copy(
                    qt_h.at[di_all.at[pl.ds(off, CHUNK)]], qr[b], gsem[b])
                pltpu.async_copy(
                    kvt_h.at[si_all.at[pl.ds(off, CHUNK)]], kr[b], gsem[b])

            def process(cur, b):
                off = cur * CHUNK
                pltpu.make_async_copy(
                    qt_h.at[di_all.at[pl.ds(off, CHUNK)]], qr[b],
                    gsem[b]).wait()
                pltpu.make_async_copy(
                    kvt_h.at[si_all.at[pl.ds(off, CHUNK)]], kr[b],
                    gsem[b]).wait()

                def row(r4, carry2):
                    for u in range(4):
                        for j in range(HID // 16):
                            sl = pl.ds(j * 16, 16)
                            kr[b][r4 * 4 + u, sl] = (kr[b][r4 * 4 + u, sl]
                                                     * qr[b][r4 * 4 + u, sl])
                    return carry2

                lax.fori_loop(0, CHUNK // 4, row, 0)
                base = base0 + cur * CHUNK
                return pltpu.async_copy(kr[b], mout_h.at[pl.ds(base, CHUNK)],
                                        osem[b])

            issue(0, 0)
            issue(1, 1)
            issue(2, 2)

            def triple(i, carry):
                g = i * 3
                for b in range(3):
                    cur = g + b
                    w = process(cur, b)
                    w.wait()
                    issue(cur + 3, b)
                return carry

            lax.fori_loop(0, (NCHUNK - 4) // 3, triple, 0)
            process(NCHUNK - 4, 0).wait()
            issue(NCHUNK - 1, 0)
            process(NCHUNK - 3, 1).wait()
            process(NCHUNK - 2, 2).wait()
            process(NCHUNK - 1, 0).wait()

        _sc_cache["gather3"] = body
    return _sc_cache["gather3"](qt, kvt, di, si)


def _sc_scatter(msg_ext, di, zeros_acc):
    if "scatter" not in _sc_cache:
        @functools.partial(
            pl.kernel,
            out_type=jax.ShapeDtypeStruct((SC_CORES * SEG_HALF, TW), jnp.float32),
            mesh=_sc_mesh(),
            scratch_types=[
                [pltpu.VMEM((SCHUNK,), jnp.int32) for _ in range(2)],
                [pltpu.VMEM((SCHUNK,), jnp.int32) for _ in range(2)],
                [pltpu.VMEM((SCHUNK, TW), jnp.float32) for _ in range(2)],
                pltpu.VMEM_SHARED((ACC_ROWS, TW), jnp.float32),
                [pltpu.SemaphoreType.DMA for _ in range(2)],
                [pltpu.SemaphoreType.DMA for _ in range(2)],
            ],
        )
        def body(msg_h, di_h, zeros_h, out_h, di_v, dloc_v, msg_v, acc_sh,
                 rsem, ssem):
            cid = lax.axis_index("c")
            sid = lax.axis_index("s")
            zrow0 = sid * ZROWS_PER_TILE
            lo = cid * SEG_HALF
            base0 = sid * NCH_SC * SCHUNK

            pltpu.sync_copy(zeros_h.at[pl.ds(zrow0, ZROWS_PER_TILE)],
                            acc_sh.at[pl.ds(zrow0, ZROWS_PER_TILE)])
            plsc.subcore_barrier()

            def issue(cur, b):
                base = base0 + cur * SCHUNK
                pltpu.async_copy(di_h.at[pl.ds(base, SCHUNK)], di_v[b], rsem[b])
                pltpu.async_copy(msg_h.at[pl.ds(base, SCHUNK)], msg_v[b], rsem[b])

            def process(cur, b):
                base = base0 + cur * SCHUNK
                pltpu.make_async_copy(di_h.at[pl.ds(base, SCHUNK)], di_v[b],
                                      rsem[b]).wait()
                pltpu.make_async_copy(msg_h.at[pl.ds(base, SCHUNK)], msg_v[b],
                                      rsem[b]).wait()
                for j in range(SCHUNK // 16):
                    sl = pl.ds(j * 16, 16)
                    d = di_v[b][sl] - lo
                    ok = (d >= 0) & (d < SEG_HALF)
                    dloc_v[b][sl] = jnp.where(ok, d, SEG_HALF)
                return pltpu.async_copy(msg_v[b], acc_sh.at[dloc_v[b]],
                                        ssem[b], add=True)

            issue(0, 0)
            issue(1, 1)

            def pair(i, carry):
                g = i * 2
                for b in range(2):
                    cur = g + b
                    w = process(cur, b)
                    w.wait()
                    issue(cur + 2, b)
                return carry

            lax.fori_loop(0, (NCH_SC - 2) // 2, pair, 0)
            process(NCH_SC - 2, 0).wait()
            process(NCH_SC - 1, 1).wait()
            plsc.subcore_barrier()

            row0 = sid * HALF_PER_TILE
            pltpu.sync_copy(acc_sh.at[pl.ds(row0, HALF_PER_TILE)],
                            out_h.at[pl.ds(cid * SEG_HALF + row0, HALF_PER_TILE)])

        _sc_cache["scatter"] = body
    return _sc_cache["scatter"](msg_ext, di, zeros_acc).reshape(
        SC_CORES, SEG_HALF, TW)


def _blockdiag(rel):
    return jax.scipy.linalg.block_diag(*[rel[h] for h in range(HEADS)])


def _pad_idx(idx):
    return jnp.concatenate([idx, jnp.zeros((EDGE_PAD - N_EDGES,), jnp.int32)])


def kernel(x_constraint, x_variable, edge_index_vc, edge_index_cv,
           batch_constraint, batch_variable, params):
    f32 = jnp.float32

    eye_h = np.zeros((HID, HEADS), np.float32)
    for h in range(HEADS):
        eye_h[h * DH:(h + 1) * DH, h] = 1.0
    head_sum_pad = jnp.asarray(
        np.concatenate([eye_h, np.zeros((TW - HID, HEADS), np.float32)]))
    head_rep = jnp.asarray(eye_h.T)
    selm = np.zeros((TW, HID), np.float32)
    selm[:HID, :HID] = np.eye(HID, dtype=np.float32)
    sel_msg = jnp.asarray(selm)
    seld = np.zeros((TW, HID), np.float32)
    for h in range(HEADS):
        seld[HID + h, h * DH:(h + 1) * DH] = 1.0
    sel_den = jnp.asarray(seld)

    zeros_acc = jnp.zeros((ACC_ROWS, TW), f32)

    si_vc = _pad_idx(edge_index_vc[0])
    di_vc = _pad_idx(edge_index_vc[1])
    si_cv = _pad_idx(edge_index_cv[0])
    di_cv = _pad_idx(edge_index_cv[1])

    h_c = _tc_linear(x_constraint, params["lin_dict"]["constraint"]["W"],
                     params["lin_dict"]["constraint"]["b"], "relu")
    h_v = _tc_linear(x_variable, params["lin_dict"]["variable"]["W"],
                     params["lin_dict"]["variable"]["b"], "relu")

    vc = "variable__to__constraint"
    cv = "constraint__to__variable"

    for p in params["convs"]:
        A_cv = _blockdiag(p["a_rel"][cv])
        M_cv = _blockdiag(p["m_rel"][cv])
        A_vc = _blockdiag(p["a_rel"][vc])
        M_vc = _blockdiag(p["m_rel"][vc])

        qt_c, kvt_c = _tc_qkv(
            h_c,
            p["q_lin"]["constraint"]["W"], p["q_lin"]["constraint"]["b"],
            p["k_lin"]["constraint"]["W"] @ A_cv, p["k_lin"]["constraint"]["b"] @ A_cv,
            p["v_lin"]["constraint"]["W"] @ M_cv, p["v_lin"]["constraint"]["b"] @ M_cv)
        qt_v, kvt_v = _tc_qkv(
            h_v,
            p["q_lin"]["variable"]["W"], p["q_lin"]["variable"]["b"],
            p["k_lin"]["variable"]["W"] @ A_vc, p["k_lin"]["variable"]["b"] @ A_vc,
            p["v_lin"]["variable"]["W"] @ M_vc, p["v_lin"]["variable"]["b"] @ M_vc)

        mout_vc = _sc_gather3(qt_c, kvt_v, di_vc, si_vc)
        mout_cv = _sc_gather3(qt_v, kvt_c, di_cv, si_cv)
        ps_vc = (p["p_rel"][vc] / np.sqrt(DH)).reshape(1, HEADS)
        ps_cv = (p["p_rel"][cv] / np.sqrt(DH)).reshape(1, HEADS)
        lg_vc, gm_vc = _tc_logits(mout_vc, head_sum_pad, ps_vc)
        msg_vc = _tc_msg(lg_vc, gm_vc, mout_vc, head_rep)
        parts_vc = _sc_scatter(msg_vc, di_vc, zeros_acc)
        lg_cv, gm_cv = _tc_logits(mout_cv, head_sum_pad, ps_cv)
        msg_cv = _tc_msg(lg_cv, gm_cv, mout_cv, head_rep)
        parts_cv = _sc_scatter(msg_cv, di_cv, zeros_acc)
        hc_pad = jnp.pad(h_c, ((0, SEG_PAD - N_NODES), (0, 0)))
        hv_pad = jnp.pad(h_v, ((0, SEG_PAD - N_NODES), (0, 0)))
        h_c = _tc_combine(parts_vc, hc_pad,
                          p["a_lin"]["constraint"]["W"],
                          p["a_lin"]["constraint"]["b"],
                          p["skip"]["constraint"], sel_msg, sel_den)[:N_NODES]
        h_v = _tc_combine(parts_cv, hv_pad,
                          p["a_lin"]["variable"]["W"],
                          p["a_lin"]["variable"]["b"],
                          p["skip"]["variable"], sel_msg, sel_den)[:N_NODES]

    pooled = _tc_pool(h_c, h_v,
                      batch_constraint.reshape(-1, 1),
                      batch_variable.reshape(-1, 1),
                      params["lin"]["W"], params["lin"]["b"])
    return pooled[:, :2]
